# baseline (speedup 1.0000x reference)
import functools

import jax
import jax.numpy as jnp
from jax.experimental import pallas as pl

_HI = jax.lax.Precision.HIGHEST


def _passA_body(a_ref, z_ref, ztb_ref, t_ref, u0_ref, u1_ref, xz_ref, *, rblk):
    i = pl.program_id(0)
    t = t_ref[...]
    u0_ref[...] = jnp.dot(a_ref[0], t, precision=_HI)
    u1_ref[...] = jnp.dot(a_ref[1], t, precision=_HI)
    r = jnp.dot(z_ref[...], ztb_ref[...], precision=_HI)
    r = jnp.maximum(r, 0.0)
    m = jnp.max(r, axis=0)
    e = jnp.exp(r - m[None, :])
    s = jnp.sum(e, axis=0)
    tb = t_ref[pl.ds(i * rblk, rblk), :] / s[:, None]
    contrib = jnp.dot(e, tb, precision=_HI)

    @pl.when(i == 0)
    def _init():
        xz_ref[...] = jnp.zeros_like(xz_ref)

    xz_ref[...] += contrib


def _passB_body(a_ref, u0_ref, u1_ref, xz_ref, t_ref, w_ref, y_ref, acc_ref,
                *, rblk):
    i = pl.program_id(0)
    v0 = jnp.dot(a_ref[0], u0_ref[...], precision=_HI)
    v1 = jnp.dot(a_ref[1], u1_ref[...], precision=_HI)
    sl = pl.ds(i * rblk, rblk)
    xs = (u0_ref[sl, :] + 2.0 * v0 + u1_ref[sl, :] + 2.0 * v1
          + xz_ref[sl, :] - t_ref[sl, :])
    y = jnp.maximum(jnp.dot(xs, w_ref[...], precision=_HI), 0.0)
    y_ref[...] = y

    @pl.when(i == 0)
    def _init():
        acc_ref[...] = jnp.zeros_like(acc_ref)

    acc_ref[...] += jnp.sum(y, axis=0, keepdims=True)


def _passA(A, Z, Zt, T, *, rblk, interpret=False):
    n, C = T.shape
    dz = Z.shape[1]
    nblk = n // rblk
    grid = (nblk,)
    return pl.pallas_call(
        functools.partial(_passA_body, rblk=rblk),
        grid=grid,
        in_specs=[
            pl.BlockSpec((2, rblk, n), lambda i: (0, i, 0)),
            pl.BlockSpec((n, dz), lambda i: (0, 0)),
            pl.BlockSpec((dz, rblk), lambda i: (0, i)),
            pl.BlockSpec((n, C), lambda i: (0, 0)),
        ],
        out_specs=[
            pl.BlockSpec((rblk, C), lambda i: (i, 0)),
            pl.BlockSpec((rblk, C), lambda i: (i, 0)),
            pl.BlockSpec((n, C), lambda i: (0, 0)),
        ],
        out_shape=[
            jax.ShapeDtypeStruct((n, C), jnp.float32),
            jax.ShapeDtypeStruct((n, C), jnp.float32),
            jax.ShapeDtypeStruct((n, C), jnp.float32),
        ],
        interpret=interpret,
    )(A, Z, Zt, T)


def _passB(A, U0, U1, Xz, T, W, *, rblk, interpret=False):
    n, C = T.shape
    nblk = n // rblk
    grid = (nblk,)
    return pl.pallas_call(
        functools.partial(_passB_body, rblk=rblk),
        grid=grid,
        in_specs=[
            pl.BlockSpec((2, rblk, n), lambda i: (0, i, 0)),
            pl.BlockSpec((n, C), lambda i: (0, 0)),
            pl.BlockSpec((n, C), lambda i: (0, 0)),
            pl.BlockSpec((n, C), lambda i: (0, 0)),
            pl.BlockSpec((n, C), lambda i: (0, 0)),
            pl.BlockSpec((C, C), lambda i: (0, 0)),
        ],
        out_specs=[
            pl.BlockSpec((rblk, C), lambda i: (i, 0)),
            pl.BlockSpec((1, C), lambda i: (0, 0)),
        ],
        out_shape=[
            jax.ShapeDtypeStruct((n, C), jnp.float32),
            jax.ShapeDtypeStruct((1, C), jnp.float32),
        ],
        interpret=interpret,
    )(A, U0, U1, Xz, T, W)


def _gwnet(A, X, Z, W1, W2, *, rblk=256, interpret=False):
    B, d, n = X.shape
    h1 = W2.shape[1]
    T1 = X.reshape(B * d, n).T
    Zt = Z.T
    Wbd1 = jnp.kron(jnp.eye(B, dtype=W1.dtype), W1)
    Wbd2 = jnp.kron(jnp.eye(B, dtype=W2.dtype), W2)

    U0, U1, Xz = _passA(A, Z, Zt, T1, rblk=rblk, interpret=interpret)
    T2, _ = _passB(A, U0, U1, Xz, T1, Wbd1, rblk=rblk, interpret=interpret)
    U0, U1, Xz = _passA(A, Z, Zt, T2, rblk=rblk, interpret=interpret)
    _, acc = _passB(A, U0, U1, Xz, T2, Wbd2, rblk=rblk, interpret=interpret)
    return (acc / n).reshape(B, h1)


def kernel(A, X, Z, W1, W2):
    return _gwnet(A, X, Z, W1, W2)


# speedup vs baseline: 2.6161x; 2.6161x over previous
import functools

import jax
import jax.numpy as jnp
from jax.experimental import pallas as pl

_HI = jax.lax.Precision.HIGHEST
_DEF = jax.lax.Precision.DEFAULT


def _passA_body(a_ref, z_ref, ztb_ref, t_ref, u0_ref, u1_ref, xz_ref, *, rblk):
    i = pl.program_id(0)
    t = t_ref[...]
    u0_ref[...] = jnp.dot(a_ref[0], t, precision=_DEF)
    u1_ref[...] = jnp.dot(a_ref[1], t, precision=_DEF)
    r = jnp.dot(z_ref[...], ztb_ref[...], precision=_HI)
    r = jnp.maximum(r, 0.0)
    m = jnp.max(r, axis=0)
    e = jnp.exp(r - m[None, :])
    s = jnp.sum(e, axis=0)
    tb = t_ref[pl.ds(i * rblk, rblk), :] / s[:, None]
    contrib = jnp.dot(e, tb, precision=_DEF)

    @pl.when(i == 0)
    def _init():
        xz_ref[...] = jnp.zeros_like(xz_ref)

    xz_ref[...] += contrib


def _passB_body(a_ref, u0_ref, u1_ref, xz_ref, t_ref, w_ref, y_ref, acc_ref,
                *, rblk):
    i = pl.program_id(0)
    v0 = jnp.dot(a_ref[0], u0_ref[...], precision=_DEF)
    v1 = jnp.dot(a_ref[1], u1_ref[...], precision=_DEF)
    sl = pl.ds(i * rblk, rblk)
    xs = (u0_ref[sl, :] + 2.0 * v0 + u1_ref[sl, :] + 2.0 * v1
          + xz_ref[sl, :] - t_ref[sl, :])
    y = jnp.maximum(jnp.dot(xs, w_ref[...], precision=_HI), 0.0)
    y_ref[...] = y

    @pl.when(i == 0)
    def _init():
        acc_ref[...] = jnp.zeros_like(acc_ref)

    acc_ref[...] += jnp.sum(y, axis=0, keepdims=True)


def _passA(A, Z, Zt, T, *, rblk, interpret=False):
    n, C = T.shape
    dz = Z.shape[1]
    nblk = n // rblk
    grid = (nblk,)
    return pl.pallas_call(
        functools.partial(_passA_body, rblk=rblk),
        grid=grid,
        in_specs=[
            pl.BlockSpec((2, rblk, n), lambda i: (0, i, 0)),
            pl.BlockSpec((n, dz), lambda i: (0, 0)),
            pl.BlockSpec((dz, rblk), lambda i: (0, i)),
            pl.BlockSpec((n, C), lambda i: (0, 0)),
        ],
        out_specs=[
            pl.BlockSpec((rblk, C), lambda i: (i, 0)),
            pl.BlockSpec((rblk, C), lambda i: (i, 0)),
            pl.BlockSpec((n, C), lambda i: (0, 0)),
        ],
        out_shape=[
            jax.ShapeDtypeStruct((n, C), jnp.float32),
            jax.ShapeDtypeStruct((n, C), jnp.float32),
            jax.ShapeDtypeStruct((n, C), jnp.float32),
        ],
        interpret=interpret,
    )(A, Z, Zt, T)


def _passB(A, U0, U1, Xz, T, W, *, rblk, interpret=False):
    n, C = T.shape
    nblk = n // rblk
    grid = (nblk,)
    return pl.pallas_call(
        functools.partial(_passB_body, rblk=rblk),
        grid=grid,
        in_specs=[
            pl.BlockSpec((2, rblk, n), lambda i: (0, i, 0)),
            pl.BlockSpec((n, C), lambda i: (0, 0)),
            pl.BlockSpec((n, C), lambda i: (0, 0)),
            pl.BlockSpec((n, C), lambda i: (0, 0)),
            pl.BlockSpec((n, C), lambda i: (0, 0)),
            pl.BlockSpec((C, C), lambda i: (0, 0)),
        ],
        out_specs=[
            pl.BlockSpec((rblk, C), lambda i: (i, 0)),
            pl.BlockSpec((1, C), lambda i: (0, 0)),
        ],
        out_shape=[
            jax.ShapeDtypeStruct((n, C), jnp.float32),
            jax.ShapeDtypeStruct((1, C), jnp.float32),
        ],
        interpret=interpret,
    )(A, U0, U1, Xz, T, W)


def _gwnet(A, X, Z, W1, W2, *, rblk=256, interpret=False):
    B, d, n = X.shape
    h1 = W2.shape[1]
    T1 = X.reshape(B * d, n).T
    Zt = Z.T
    Wbd1 = jnp.kron(jnp.eye(B, dtype=W1.dtype), W1)
    Wbd2 = jnp.kron(jnp.eye(B, dtype=W2.dtype), W2)

    U0, U1, Xz = _passA(A, Z, Zt, T1, rblk=rblk, interpret=interpret)
    T2, _ = _passB(A, U0, U1, Xz, T1, Wbd1, rblk=rblk, interpret=interpret)
    U0, U1, Xz = _passA(A, Z, Zt, T2, rblk=rblk, interpret=interpret)
    _, acc = _passB(A, U0, U1, Xz, T2, Wbd2, rblk=rblk, interpret=interpret)
    return (acc / n).reshape(B, h1)


def kernel(A, X, Z, W1, W2):
    return _gwnet(A, X, Z, W1, W2)


# speedup vs baseline: 3.6496x; 1.3951x over previous
import functools

import jax
import jax.numpy as jnp
from jax.experimental import pallas as pl
from jax.experimental.pallas import tpu as pltpu

_DEF = jax.lax.Precision.DEFAULT


def _body(a_ref, z_hbm, zt_hbm, t1_hbm, w1_hbm, w2_hbm, acc_ref,
          z_s, zt_s, t1_s, w1_s, w2_s, u0_s, u1_s, xz_s, t2_s, sem,
          *, rblk, nblk):
    p = pl.program_id(0)
    i = pl.program_id(1)

    @pl.when((p == 0) & (i == 0))
    def _prologue():
        for src, dst in ((z_hbm, z_s), (zt_hbm, zt_s), (t1_hbm, t1_s),
                         (w1_hbm, w1_s), (w2_hbm, w2_s)):
            cp = pltpu.make_async_copy(src, dst, sem)
            cp.start()
            cp.wait()
        acc_ref[...] = jnp.zeros_like(acc_ref)

    sl = pl.ds(i * rblk, rblk)

    def even_phase(t_s):
        u0_s[sl, :] = jnp.dot(a_ref[0], t_s[...], precision=_DEF)
        u1_s[sl, :] = jnp.dot(a_ref[1], t_s[...], precision=_DEF)
        r = jnp.dot(z_s[...], zt_s[:, sl], precision=_DEF)
        r = jnp.maximum(r, 0.0)
        m = jnp.max(r, axis=0)
        e = jnp.exp(r - m[None, :])
        s = jnp.sum(e, axis=0)
        tb = t_s[sl, :] / s[:, None]
        contrib = jnp.dot(e, tb, precision=_DEF)

        @pl.when(i == 0)
        def _init():
            xz_s[...] = jnp.zeros_like(xz_s)

        xz_s[...] += contrib

    def odd_phase(t_s, w_s, is_final):
        v0 = jnp.dot(a_ref[0], u0_s[...], precision=_DEF)
        v1 = jnp.dot(a_ref[1], u1_s[...], precision=_DEF)
        xs = (u0_s[sl, :] + 2.0 * v0 + u1_s[sl, :] + 2.0 * v1
              + xz_s[sl, :] - t_s[sl, :])
        y = jnp.maximum(jnp.dot(xs, w_s[...], precision=_DEF), 0.0)
        if is_final:
            acc_ref[...] += jnp.sum(y, axis=0, keepdims=True)
        else:
            t2_s[sl, :] = y

    @pl.when(p == 0)
    def _p0():
        even_phase(t1_s)

    @pl.when(p == 1)
    def _p1():
        odd_phase(t1_s, w1_s, False)

    @pl.when(p == 2)
    def _p2():
        even_phase(t2_s)

    @pl.when(p == 3)
    def _p3():
        odd_phase(t2_s, w2_s, True)


def _gwnet(A, X, Z, W1, W2, *, rblk=256, interpret=False):
    B, d, n = X.shape
    dz = Z.shape[1]
    h1 = W2.shape[1]
    C = B * d
    nblk = n // rblk
    T1 = X.reshape(C, n).T
    Zt = Z.T
    Wbd1 = jnp.kron(jnp.eye(B, dtype=W1.dtype), W1)
    Wbd2 = jnp.kron(jnp.eye(B, dtype=W2.dtype), W2)

    f32 = jnp.float32
    acc = pl.pallas_call(
        functools.partial(_body, rblk=rblk, nblk=nblk),
        grid=(4, nblk),
        in_specs=[
            pl.BlockSpec((2, rblk, n), lambda p, i: (0, i, 0)),
            pl.BlockSpec(memory_space=pl.ANY),
            pl.BlockSpec(memory_space=pl.ANY),
            pl.BlockSpec(memory_space=pl.ANY),
            pl.BlockSpec(memory_space=pl.ANY),
            pl.BlockSpec(memory_space=pl.ANY),
        ],
        out_specs=pl.BlockSpec((1, C), lambda p, i: (0, 0)),
        out_shape=jax.ShapeDtypeStruct((1, C), f32),
        scratch_shapes=[
            pltpu.VMEM((n, dz), f32),
            pltpu.VMEM((dz, n), f32),
            pltpu.VMEM((n, C), f32),
            pltpu.VMEM((C, C), f32),
            pltpu.VMEM((C, C), f32),
            pltpu.VMEM((n, C), f32),
            pltpu.VMEM((n, C), f32),
            pltpu.VMEM((n, C), f32),
            pltpu.VMEM((n, C), f32),
            pltpu.SemaphoreType.DMA,
        ],
        interpret=interpret,
    )(A, Z, Zt, T1, Wbd1, Wbd2)
    return (acc / n).reshape(B, h1)


def kernel(A, X, Z, W1, W2):
    return _gwnet(A, X, Z, W1, W2)


# speedup vs baseline: 4.6079x; 1.2626x over previous
import functools

import jax
import jax.numpy as jnp
from jax.experimental import pallas as pl
from jax.experimental.pallas import tpu as pltpu

_F32 = jnp.float32
_BF16 = jnp.bfloat16


def _dot(a, b):
    return jnp.dot(a, b, preferred_element_type=_F32)


def _k1_body(a_ref, zb_hbm, ztb_hbm, t1_hbm,
             abf_ref, u0_ref, u1_ref, xz_hbm, s_hbm,
             zb_s, ztb_s, t1_s, t1bf_s, xz_s, s_s, sem,
             *, rblk, nblk):
    i = pl.program_id(0)
    sl = pl.ds(i * rblk, rblk)

    @pl.when(i == 0)
    def _prologue():
        for src, dst in ((zb_hbm, zb_s), (ztb_hbm, ztb_s), (t1_hbm, t1_s)):
            cp = pltpu.make_async_copy(src, dst, sem)
            cp.start()
            cp.wait()
        t1bf_s[...] = t1_s[...].astype(_BF16)
        xz_s[...] = jnp.zeros_like(xz_s)

    a0b = a_ref[0].astype(_BF16)
    a1b = a_ref[1].astype(_BF16)
    abf_ref[0] = a0b
    abf_ref[1] = a1b
    u0_ref[...] = _dot(a0b, t1bf_s[...])
    u1_ref[...] = _dot(a1b, t1bf_s[...])
    r = _dot(zb_s[...], ztb_s[:, sl])
    e = jnp.exp(jnp.maximum(r, 0.0))
    s = jnp.sum(e, axis=0)
    s_s[0, sl] = s
    ebf = (e * (1.0 / s)[None, :]).astype(_BF16)
    xz_s[...] += _dot(ebf, t1bf_s[sl, :])

    @pl.when(i == nblk - 1)
    def _epilogue():
        for src, dst in ((xz_s, xz_hbm), (s_s, s_hbm)):
            cp = pltpu.make_async_copy(src, dst, sem)
            cp.start()
            cp.wait()


def _k2_body(abf_ref, u0_hbm, u1_hbm, xz_hbm, t1_hbm, s_hbm,
             zb_hbm, ztb_hbm, w1_hbm, w2_hbm,
             acc_ref,
             u0_s, u1_s, xz_s, t1_s, s_s, zb_s, ztb_s, w1_s, w2_s,
             q0_s, q1_s, t2_s, t2bf_s, sem,
             *, rblk, nblk):
    q = pl.program_id(0)
    i = pl.program_id(1)
    sl = pl.ds(i * rblk, rblk)

    @pl.when((q == 0) & (i == 0))
    def _prologue():
        for src, dst in ((u0_hbm, u0_s), (u1_hbm, u1_s), (xz_hbm, xz_s),
                         (t1_hbm, t1_s), (s_hbm, s_s), (zb_hbm, zb_s),
                         (ztb_hbm, ztb_s), (w1_hbm, w1_s), (w2_hbm, w2_s)):
            cp = pltpu.make_async_copy(src, dst, sem)
            cp.start()
            cp.wait()
        q0_s[...] = (t1_s[...] + 2.0 * u0_s[...]).astype(_BF16)
        q1_s[...] = (t1_s[...] + 2.0 * u1_s[...]).astype(_BF16)
        acc_ref[...] = jnp.zeros_like(acc_ref)

    @pl.when(q == 0)
    def _l1_sweep2():
        s0 = _dot(abf_ref[0], q0_s[...])
        s1 = _dot(abf_ref[1], q1_s[...])
        xs = s0 + s1 + xz_s[sl, :] - t1_s[sl, :]
        t2_s[sl, :] = jnp.maximum(jnp.dot(xs, w1_s[...]), 0.0)

        @pl.when(i == nblk - 1)
        def _finish():
            t2bf_s[...] = t2_s[...].astype(_BF16)

    @pl.when(q == 1)
    def _l2_sweep1():
        @pl.when(i == 0)
        def _init():
            xz_s[...] = jnp.zeros_like(xz_s)

        u0_s[sl, :] = _dot(abf_ref[0], t2bf_s[...])
        u1_s[sl, :] = _dot(abf_ref[1], t2bf_s[...])
        r = _dot(zb_s[...], ztb_s[:, sl])
        e = jnp.exp(jnp.maximum(r, 0.0))
        ebf = (e * (1.0 / s_s[0, sl])[None, :]).astype(_BF16)
        xz_s[...] += _dot(ebf, t2bf_s[sl, :])

        @pl.when(i == nblk - 1)
        def _finish():
            q0_s[...] = (t2_s[...] + 2.0 * u0_s[...]).astype(_BF16)
            q1_s[...] = (t2_s[...] + 2.0 * u1_s[...]).astype(_BF16)

    @pl.when(q == 2)
    def _l2_sweep2():
        s0 = _dot(abf_ref[0], q0_s[...])
        s1 = _dot(abf_ref[1], q1_s[...])
        xs = s0 + s1 + xz_s[sl, :] - t2_s[sl, :]
        y = jnp.maximum(jnp.dot(xs, w2_s[...]), 0.0)
        acc_ref[...] += jnp.sum(y, axis=0, keepdims=True)


def _gwnet(A, X, Z, W1, W2, *, rblk=256, interpret=False):
    B, d, n = X.shape
    dz = Z.shape[1]
    h1 = W2.shape[1]
    C = B * d
    nblk = n // rblk
    T1 = X.reshape(C, n).T
    Zb = Z.astype(_BF16)
    Ztb = Zb.T
    Wbd1 = jnp.kron(jnp.eye(B, dtype=W1.dtype), W1)
    Wbd2 = jnp.kron(jnp.eye(B, dtype=W2.dtype), W2)

    any_spec = pl.BlockSpec(memory_space=pl.ANY)

    Abf, U0, U1, Xz, s = pl.pallas_call(
        functools.partial(_k1_body, rblk=rblk, nblk=nblk),
        grid=(nblk,),
        in_specs=[
            pl.BlockSpec((2, rblk, n), lambda i: (0, i, 0)),
            any_spec, any_spec, any_spec,
        ],
        out_specs=[
            pl.BlockSpec((2, rblk, n), lambda i: (0, i, 0)),
            pl.BlockSpec((rblk, C), lambda i: (i, 0)),
            pl.BlockSpec((rblk, C), lambda i: (i, 0)),
            any_spec,
            any_spec,
        ],
        out_shape=[
            jax.ShapeDtypeStruct((2, n, n), _BF16),
            jax.ShapeDtypeStruct((n, C), _F32),
            jax.ShapeDtypeStruct((n, C), _F32),
            jax.ShapeDtypeStruct((n, C), _F32),
            jax.ShapeDtypeStruct((1, n), _F32),
        ],
        scratch_shapes=[
            pltpu.VMEM((n, dz), _BF16),
            pltpu.VMEM((dz, n), _BF16),
            pltpu.VMEM((n, C), _F32),
            pltpu.VMEM((n, C), _BF16),
            pltpu.VMEM((n, C), _F32),
            pltpu.VMEM((1, n), _F32),
            pltpu.SemaphoreType.DMA,
        ],
        interpret=interpret,
    )(A, Zb, Ztb, T1)

    acc = pl.pallas_call(
        functools.partial(_k2_body, rblk=rblk, nblk=nblk),
        grid=(3, nblk),
        in_specs=[
            pl.BlockSpec((2, rblk, n), lambda q, i: (0, i, 0)),
            any_spec, any_spec, any_spec, any_spec, any_spec,
            any_spec, any_spec, any_spec, any_spec,
        ],
        out_specs=pl.BlockSpec((1, C), lambda q, i: (0, 0)),
        out_shape=jax.ShapeDtypeStruct((1, C), _F32),
        scratch_shapes=[
            pltpu.VMEM((n, C), _F32),
            pltpu.VMEM((n, C), _F32),
            pltpu.VMEM((n, C), _F32),
            pltpu.VMEM((n, C), _F32),
            pltpu.VMEM((1, n), _F32),
            pltpu.VMEM((n, dz), _BF16),
            pltpu.VMEM((dz, n), _BF16),
            pltpu.VMEM((C, C), _F32),
            pltpu.VMEM((C, C), _F32),
            pltpu.VMEM((n, C), _BF16),
            pltpu.VMEM((n, C), _BF16),
            pltpu.VMEM((n, C), _F32),
            pltpu.VMEM((n, C), _BF16),
            pltpu.SemaphoreType.DMA,
        ],
        interpret=interpret,
    )(Abf, U0, U1, Xz, T1, s, Zb, Ztb, Wbd1, Wbd2)
    return (acc / n).reshape(B, h1)


def kernel(A, X, Z, W1, W2):
    return _gwnet(A, X, Z, W1, W2)


# speedup vs baseline: 5.2090x; 1.1304x over previous
import functools

import jax
import jax.numpy as jnp
from jax.experimental import pallas as pl
from jax.experimental.pallas import tpu as pltpu

_F32 = jnp.float32
_BF16 = jnp.bfloat16


def _dot(a, b):
    return jnp.dot(a, b, preferred_element_type=_F32)


def _copy_all(pairs, sem):
    copies = [pltpu.make_async_copy(src, dst, sem) for src, dst in pairs]
    for cp in copies:
        cp.start()
    for cp in copies:
        cp.wait()


def _k1_body(a_ref, zb_hbm, ztb_hbm, t1_hbm,
             abf_ref, q0_ref, q1_ref, xz_hbm, s_hbm,
             zb_s, ztb_s, t1_s, t1bf_s, xz_s, s_s, sem,
             *, rblk, nblk):
    i = pl.program_id(0)
    sl = pl.ds(i * rblk, rblk)

    @pl.when(i == 0)
    def _prologue():
        _copy_all(((zb_hbm, zb_s), (ztb_hbm, ztb_s), (t1_hbm, t1_s)), sem)
        t1bf_s[...] = t1_s[...].astype(_BF16)
        xz_s[...] = jnp.zeros_like(xz_s)

    a0b = a_ref[0].astype(_BF16)
    a1b = a_ref[1].astype(_BF16)
    abf_ref[0] = a0b
    abf_ref[1] = a1b
    u0 = _dot(a0b, t1bf_s[...])
    u1 = _dot(a1b, t1bf_s[...])
    t1blk = t1_s[sl, :]
    q0_ref[...] = (t1blk + 2.0 * u0).astype(_BF16)
    q1_ref[...] = (t1blk + 2.0 * u1).astype(_BF16)
    r = _dot(zb_s[...], ztb_s[:, sl])
    e = jnp.exp(jnp.maximum(r, 0.0))
    s = jnp.sum(e, axis=0)
    s_s[0, sl] = s
    ebf = (e * (1.0 / s)[None, :]).astype(_BF16)
    xz_s[...] += _dot(ebf, t1bf_s[sl, :])

    @pl.when(i == nblk - 1)
    def _epilogue():
        _copy_all(((xz_s, xz_hbm), (s_s, s_hbm)), sem)


def _k2_body(abf_ref, q0_hbm, q1_hbm, xz_hbm, t1_hbm, s_hbm,
             zb_hbm, ztb_hbm, w1_hbm, w2_hbm,
             acc_ref,
             u0_s, u1_s, xz_s, t1_s, s_s, zb_s, ztb_s, w1_s, w2_s,
             q0_s, q1_s, t2_s, t2bf_s, sem,
             *, rblk, nblk, cblk):
    q = pl.program_id(0)
    i = pl.program_id(1)
    sl = pl.ds(i * rblk, rblk)

    @pl.when((q == 0) & (i == 0))
    def _prologue():
        _copy_all(((q0_hbm, q0_s), (q1_hbm, q1_s), (xz_hbm, xz_s),
                   (t1_hbm, t1_s), (s_hbm, s_s), (zb_hbm, zb_s),
                   (ztb_hbm, ztb_s), (w1_hbm, w1_s), (w2_hbm, w2_s)), sem)
        acc_ref[...] = jnp.zeros_like(acc_ref)

    @pl.when(q == 0)
    def _l1_sweep2():
        s0 = _dot(abf_ref[0], q0_s[...])
        s1 = _dot(abf_ref[1], q1_s[...])
        xs = s0 + s1 + xz_s[sl, :] - t1_s[sl, :]
        t2_s[sl, :] = jnp.maximum(jnp.dot(xs, w1_s[...]), 0.0)

        @pl.when(i == nblk - 1)
        def _finish():
            t2bf_s[...] = t2_s[...].astype(_BF16)

    @pl.when(q == 1)
    def _l2_sweep1():
        @pl.when(i == 0)
        def _init():
            xz_s[...] = jnp.zeros_like(xz_s)

        u0_s[sl, :] = _dot(abf_ref[0], t2bf_s[...])
        u1_s[sl, :] = _dot(abf_ref[1], t2bf_s[...])
        for c in range(rblk // cblk):
            csl = pl.ds(i * rblk + c * cblk, cblk)
            r = _dot(zb_s[...], ztb_s[:, csl])
            e = jnp.exp(jnp.maximum(r, 0.0))
            ebf = (e * (1.0 / s_s[0, csl])[None, :]).astype(_BF16)
            xz_s[...] += _dot(ebf, t2bf_s[csl, :])

        @pl.when(i == nblk - 1)
        def _finish():
            q0_s[...] = (t2_s[...] + 2.0 * u0_s[...]).astype(_BF16)
            q1_s[...] = (t2_s[...] + 2.0 * u1_s[...]).astype(_BF16)

    @pl.when(q == 2)
    def _l2_sweep2():
        s0 = _dot(abf_ref[0], q0_s[...])
        s1 = _dot(abf_ref[1], q1_s[...])
        xs = s0 + s1 + xz_s[sl, :] - t2_s[sl, :]
        y = jnp.maximum(jnp.dot(xs, w2_s[...]), 0.0)
        acc_ref[...] += jnp.sum(y, axis=0, keepdims=True)


def _gwnet(A, X, Z, W1, W2, *, rblk1=256, rblk2=512, interpret=False):
    B, d, n = X.shape
    dz = Z.shape[1]
    h1 = W2.shape[1]
    C = B * d
    nblk1 = n // rblk1
    nblk2 = n // rblk2
    T1 = X.reshape(C, n).T
    Zb = Z.astype(_BF16)
    Ztb = Zb.T
    Wbd1 = jnp.kron(jnp.eye(B, dtype=W1.dtype), W1)
    Wbd2 = jnp.kron(jnp.eye(B, dtype=W2.dtype), W2)

    any_spec = pl.BlockSpec(memory_space=pl.ANY)

    Abf, Q0, Q1, Xz, s = pl.pallas_call(
        functools.partial(_k1_body, rblk=rblk1, nblk=nblk1),
        grid=(nblk1,),
        in_specs=[
            pl.BlockSpec((2, rblk1, n), lambda i: (0, i, 0)),
            any_spec, any_spec, any_spec,
        ],
        out_specs=[
            pl.BlockSpec((2, rblk1, n), lambda i: (0, i, 0)),
            pl.BlockSpec((rblk1, C), lambda i: (i, 0)),
            pl.BlockSpec((rblk1, C), lambda i: (i, 0)),
            any_spec,
            any_spec,
        ],
        out_shape=[
            jax.ShapeDtypeStruct((2, n, n), _BF16),
            jax.ShapeDtypeStruct((n, C), _BF16),
            jax.ShapeDtypeStruct((n, C), _BF16),
            jax.ShapeDtypeStruct((n, C), _F32),
            jax.ShapeDtypeStruct((1, n), _F32),
        ],
        scratch_shapes=[
            pltpu.VMEM((n, dz), _BF16),
            pltpu.VMEM((dz, n), _BF16),
            pltpu.VMEM((n, C), _F32),
            pltpu.VMEM((n, C), _BF16),
            pltpu.VMEM((n, C), _F32),
            pltpu.VMEM((1, n), _F32),
            pltpu.SemaphoreType.DMA,
        ],
        interpret=interpret,
    )(A, Zb, Ztb, T1)

    acc = pl.pallas_call(
        functools.partial(_k2_body, rblk=rblk2, nblk=nblk2, cblk=rblk1),
        grid=(3, nblk2),
        in_specs=[
            pl.BlockSpec((2, rblk2, n), lambda q, i: (0, i, 0)),
            any_spec, any_spec, any_spec, any_spec, any_spec,
            any_spec, any_spec, any_spec, any_spec,
        ],
        out_specs=pl.BlockSpec((1, C), lambda q, i: (0, 0)),
        out_shape=jax.ShapeDtypeStruct((1, C), _F32),
        scratch_shapes=[
            pltpu.VMEM((n, C), _F32),
            pltpu.VMEM((n, C), _F32),
            pltpu.VMEM((n, C), _F32),
            pltpu.VMEM((n, C), _F32),
            pltpu.VMEM((1, n), _F32),
            pltpu.VMEM((n, dz), _BF16),
            pltpu.VMEM((dz, n), _BF16),
            pltpu.VMEM((C, C), _F32),
            pltpu.VMEM((C, C), _F32),
            pltpu.VMEM((n, C), _BF16),
            pltpu.VMEM((n, C), _BF16),
            pltpu.VMEM((n, C), _F32),
            pltpu.VMEM((n, C), _BF16),
            pltpu.SemaphoreType.DMA,
        ],
        interpret=interpret,
    )(Abf, Q0, Q1, Xz, T1, s, Zb, Ztb, Wbd1, Wbd2)
    return (acc / n).reshape(B, h1)


def kernel(A, X, Z, W1, W2):
    return _gwnet(A, X, Z, W1, W2)


# speedup vs baseline: 5.2110x; 1.0004x over previous
import functools

import jax
import jax.numpy as jnp
from jax.experimental import pallas as pl
from jax.experimental.pallas import tpu as pltpu

_F32 = jnp.float32
_BF16 = jnp.bfloat16


def _dot(a, b):
    return jnp.dot(a, b, preferred_element_type=_F32)


def _copy_all(pairs, sem):
    copies = [pltpu.make_async_copy(src, dst, sem) for src, dst in pairs]
    for cp in copies:
        cp.start()
    for cp in copies:
        cp.wait()


def _k1_body(a_ref, zb_hbm, ztb_hbm, t1_hbm,
             abf_ref, q0_ref, q1_ref, xz_hbm, s_hbm,
             zb_s, ztb_s, t1_s, t1bf_s, xz_s, s_s, sem,
             *, rblk, nblk, cblk):
    i = pl.program_id(0)
    sl = pl.ds(i * rblk, rblk)

    @pl.when(i == 0)
    def _prologue():
        _copy_all(((zb_hbm, zb_s), (ztb_hbm, ztb_s), (t1_hbm, t1_s)), sem)
        t1bf_s[...] = t1_s[...].astype(_BF16)
        xz_s[...] = jnp.zeros_like(xz_s)

    a0b = a_ref[0].astype(_BF16)
    a1b = a_ref[1].astype(_BF16)
    abf_ref[0] = a0b
    abf_ref[1] = a1b
    u0 = _dot(a0b, t1bf_s[...])
    u1 = _dot(a1b, t1bf_s[...])
    t1blk = t1_s[sl, :]
    q0_ref[...] = (t1blk + 2.0 * u0).astype(_BF16)
    q1_ref[...] = (t1blk + 2.0 * u1).astype(_BF16)
    for c in range(rblk // cblk):
        csl = pl.ds(i * rblk + c * cblk, cblk)
        r = _dot(zb_s[...], ztb_s[:, csl])
        e = jnp.exp(jnp.maximum(r, 0.0))
        s = jnp.sum(e, axis=0)
        s_s[0, csl] = s
        ebf = (e * (1.0 / s)[None, :]).astype(_BF16)
        xz_s[...] += _dot(ebf, t1bf_s[csl, :])

    @pl.when(i == nblk - 1)
    def _epilogue():
        _copy_all(((xz_s, xz_hbm), (s_s, s_hbm)), sem)


def _k2_body(abf_ref, q0_hbm, q1_hbm, xz_hbm, t1_hbm, s_hbm,
             zb_hbm, ztb_hbm, w1_hbm, w2_hbm,
             acc_ref,
             u0_s, u1_s, xz_s, t1_s, s_s, zb_s, ztb_s, w1_s, w2_s,
             q0_s, q1_s, t2_s, t2bf_s, sem,
             *, rblk, nblk, cblk):
    q = pl.program_id(0)
    i = pl.program_id(1)
    sl = pl.ds(i * rblk, rblk)

    @pl.when((q == 0) & (i == 0))
    def _prologue():
        _copy_all(((q0_hbm, q0_s), (q1_hbm, q1_s), (xz_hbm, xz_s),
                   (t1_hbm, t1_s), (s_hbm, s_s), (zb_hbm, zb_s),
                   (ztb_hbm, ztb_s), (w1_hbm, w1_s), (w2_hbm, w2_s)), sem)
        acc_ref[...] = jnp.zeros_like(acc_ref)

    @pl.when(q == 0)
    def _l1_sweep2():
        s0 = _dot(abf_ref[0], q0_s[...])
        s1 = _dot(abf_ref[1], q1_s[...])
        xs = s0 + s1 + xz_s[sl, :] - t1_s[sl, :]
        t2_s[sl, :] = jnp.maximum(jnp.dot(xs, w1_s[...]), 0.0)

        @pl.when(i == nblk - 1)
        def _finish():
            t2bf_s[...] = t2_s[...].astype(_BF16)

    @pl.when(q == 1)
    def _l2_sweep1():
        @pl.when(i == 0)
        def _init():
            xz_s[...] = jnp.zeros_like(xz_s)

        u0_s[sl, :] = _dot(abf_ref[0], t2bf_s[...])
        u1_s[sl, :] = _dot(abf_ref[1], t2bf_s[...])
        for c in range(rblk // cblk):
            csl = pl.ds(i * rblk + c * cblk, cblk)
            r = _dot(zb_s[...], ztb_s[:, csl])
            e = jnp.exp(jnp.maximum(r, 0.0))
            ebf = (e * (1.0 / s_s[0, csl])[None, :]).astype(_BF16)
            xz_s[...] += _dot(ebf, t2bf_s[csl, :])

        @pl.when(i == nblk - 1)
        def _finish():
            q0_s[...] = (t2_s[...] + 2.0 * u0_s[...]).astype(_BF16)
            q1_s[...] = (t2_s[...] + 2.0 * u1_s[...]).astype(_BF16)

    @pl.when(q == 2)
    def _l2_sweep2():
        s0 = _dot(abf_ref[0], q0_s[...])
        s1 = _dot(abf_ref[1], q1_s[...])
        xs = s0 + s1 + xz_s[sl, :] - t2_s[sl, :]
        y = jnp.maximum(jnp.dot(xs, w2_s[...]), 0.0)
        acc_ref[...] += jnp.sum(y, axis=0, keepdims=True)


def _gwnet(A, X, Z, W1, W2, *, rblk1=512, rblk2=512, cblk=128, interpret=False):
    B, d, n = X.shape
    dz = Z.shape[1]
    h1 = W2.shape[1]
    C = B * d
    nblk1 = n // rblk1
    nblk2 = n // rblk2
    T1 = X.reshape(C, n).T
    Zb = Z.astype(_BF16)
    Ztb = Zb.T
    Wbd1 = jnp.kron(jnp.eye(B, dtype=W1.dtype), W1)
    Wbd2 = jnp.kron(jnp.eye(B, dtype=W2.dtype), W2)

    any_spec = pl.BlockSpec(memory_space=pl.ANY)

    Abf, Q0, Q1, Xz, s = pl.pallas_call(
        functools.partial(_k1_body, rblk=rblk1, nblk=nblk1, cblk=cblk),
        grid=(nblk1,),
        in_specs=[
            pl.BlockSpec((2, rblk1, n), lambda i: (0, i, 0)),
            any_spec, any_spec, any_spec,
        ],
        out_specs=[
            pl.BlockSpec((2, rblk1, n), lambda i: (0, i, 0)),
            pl.BlockSpec((rblk1, C), lambda i: (i, 0)),
            pl.BlockSpec((rblk1, C), lambda i: (i, 0)),
            any_spec,
            any_spec,
        ],
        out_shape=[
            jax.ShapeDtypeStruct((2, n, n), _BF16),
            jax.ShapeDtypeStruct((n, C), _BF16),
            jax.ShapeDtypeStruct((n, C), _BF16),
            jax.ShapeDtypeStruct((n, C), _F32),
            jax.ShapeDtypeStruct((1, n), _F32),
        ],
        scratch_shapes=[
            pltpu.VMEM((n, dz), _BF16),
            pltpu.VMEM((dz, n), _BF16),
            pltpu.VMEM((n, C), _F32),
            pltpu.VMEM((n, C), _BF16),
            pltpu.VMEM((n, C), _F32),
            pltpu.VMEM((1, n), _F32),
            pltpu.SemaphoreType.DMA,
        ],
        interpret=interpret,
    )(A, Zb, Ztb, T1)

    acc = pl.pallas_call(
        functools.partial(_k2_body, rblk=rblk2, nblk=nblk2, cblk=2 * cblk),
        grid=(3, nblk2),
        in_specs=[
            pl.BlockSpec((2, rblk2, n), lambda q, i: (0, i, 0)),
            any_spec, any_spec, any_spec, any_spec, any_spec,
            any_spec, any_spec, any_spec, any_spec,
        ],
        out_specs=pl.BlockSpec((1, C), lambda q, i: (0, 0)),
        out_shape=jax.ShapeDtypeStruct((1, C), _F32),
        scratch_shapes=[
            pltpu.VMEM((n, C), _F32),
            pltpu.VMEM((n, C), _F32),
            pltpu.VMEM((n, C), _F32),
            pltpu.VMEM((n, C), _F32),
            pltpu.VMEM((1, n), _F32),
            pltpu.VMEM((n, dz), _BF16),
            pltpu.VMEM((dz, n), _BF16),
            pltpu.VMEM((C, C), _F32),
            pltpu.VMEM((C, C), _F32),
            pltpu.VMEM((n, C), _BF16),
            pltpu.VMEM((n, C), _BF16),
            pltpu.VMEM((n, C), _F32),
            pltpu.VMEM((n, C), _BF16),
            pltpu.SemaphoreType.DMA,
        ],
        interpret=interpret,
    )(Abf, Q0, Q1, Xz, T1, s, Zb, Ztb, Wbd1, Wbd2)
    return (acc / n).reshape(B, h1)


def kernel(A, X, Z, W1, W2):
    return _gwnet(A, X, Z, W1, W2)
